# baseline (speedup 1.0000x reference)
import jax
import jax.numpy as jnp
from jax.experimental import pallas as pl
from jax.experimental.pallas import tpu as pltpu

_NQ = 4
_NB = 4


def _body(m_hbm, a_hbm, b_hbm, o_hbm,
          m_vmem, a_vmem, b_vmem, o_vmem,
          m_sem, a_sem, b_sem, o_sem):
    H, Sq, D = a_hbm.shape
    Sk = b_hbm.shape[1]
    HQ = Sq // 2
    QB = Sq // _NQ

    def mask_copy(half):
        return pltpu.make_async_copy(
            m_hbm.at[pl.ds(half * HQ, HQ), :],
            m_vmem.at[pl.ds(half * HQ, HQ), :],
            m_sem.at[half],
        )

    def a_copy(h):
        return pltpu.make_async_copy(a_hbm.at[h], a_vmem.at[h % 2], a_sem.at[h % 2])

    def b_copy(h):
        return pltpu.make_async_copy(b_hbm.at[h], b_vmem.at[h % 2], b_sem.at[h % 2])

    def out_copy(h, quarter, buf):
        return pltpu.make_async_copy(
            o_vmem.at[buf],
            o_hbm.at[h, pl.ds(quarter * QB, QB), :],
            o_sem.at[buf],
        )

    mask_copy(0).start()
    mask_copy(1).start()
    a_copy(0).start()
    b_copy(0).start()

    for h in range(H):
        slot = h % 2
        a_copy(h).wait()
        b_copy(h).wait()
        if h + 1 < H:
            a_copy(h + 1).start()
            b_copy(h + 1).start()
        bv = b_vmem[slot]
        for quarter in range(_NQ):
            i = _NQ * h + quarter
            buf = i % _NB
            av = a_vmem[slot, pl.ds(quarter * QB, QB), :]
            acc = jax.lax.dot_general(
                av, bv, (((1,), (1,)), ((), ())),
                preferred_element_type=jnp.float32,
            )
            if h == 0 and quarter * QB % HQ == 0:
                mask_copy(quarter * QB // HQ).wait()
            if i >= _NB:
                out_copy((i - _NB) // _NQ, (i - _NB) % _NQ, buf).wait()
            m = m_vmem[pl.ds(quarter * QB, QB), :]
            o_vmem[buf] = jnp.where(m != 0, acc, jnp.float32(0.0))
            out_copy(h, quarter, buf).start()

    for i in range(_NQ * H - _NB, _NQ * H):
        out_copy(i // _NQ, i % _NQ, i % _NB).wait()


@jax.jit
def kernel(a, b, mask):
    B, H, Sq, D = a.shape
    Sk = b.shape[2]
    QB = Sq // _NQ

    a3 = a.reshape(H, Sq, D)
    b3 = b.reshape(H, Sk, D)
    m2 = mask.reshape(Sq, Sk)

    out = pl.pallas_call(
        _body,
        in_specs=[
            pl.BlockSpec(memory_space=pl.ANY),
            pl.BlockSpec(memory_space=pl.ANY),
            pl.BlockSpec(memory_space=pl.ANY),
        ],
        out_specs=pl.BlockSpec(memory_space=pl.ANY),
        out_shape=jax.ShapeDtypeStruct((H, Sq, Sk), jnp.float32),
        scratch_shapes=[
            pltpu.VMEM((Sq, Sk), jnp.int32),
            pltpu.VMEM((2, Sq, D), jnp.float32),
            pltpu.VMEM((2, Sk, D), jnp.float32),
            pltpu.VMEM((_NB, QB, Sk), jnp.float32),
            pltpu.SemaphoreType.DMA((2,)),
            pltpu.SemaphoreType.DMA((2,)),
            pltpu.SemaphoreType.DMA((2,)),
            pltpu.SemaphoreType.DMA((_NB,)),
        ],
    )(m2, a3, b3)
    return out.reshape(B, H, Sq, Sk)


# speedup vs baseline: 1.0328x; 1.0328x over previous
import jax
import jax.numpy as jnp
from jax.experimental import pallas as pl
from jax.experimental.pallas import tpu as pltpu


def _body(m_hbm, a_ref, b_ref, o_ref, m_vmem, m_sem):
    h = pl.program_id(0)

    def mask_copy():
        return pltpu.make_async_copy(m_hbm, m_vmem, m_sem)

    @pl.when(h == 0)
    def _start():
        mask_copy().start()

    bv = b_ref[0]
    Sq = a_ref.shape[1]
    QB = Sq // 4
    for quarter in range(4):
        av = a_ref[0, pl.ds(quarter * QB, QB), :]
        acc = jax.lax.dot_general(
            av, bv, (((1,), (1,)), ((), ())), preferred_element_type=jnp.float32
        )

        if quarter == 0:
            @pl.when(h == 0)
            def _wait():
                mask_copy().wait()

        m = m_vmem[pl.ds(quarter * QB, QB), :]
        o_ref[0, pl.ds(quarter * QB, QB), :] = jnp.where(
            m != 0, acc, jnp.float32(0.0)
        )


@jax.jit
def kernel(a, b, mask):
    B, H, Sq, D = a.shape
    Sk = b.shape[2]

    a3 = a.reshape(H, Sq, D)
    b3 = b.reshape(H, Sk, D)
    m2 = mask.reshape(Sq, Sk)

    out = pl.pallas_call(
        _body,
        grid=(H,),
        in_specs=[
            pl.BlockSpec(memory_space=pl.ANY),
            pl.BlockSpec((1, Sq, D), lambda h: (h, 0, 0)),
            pl.BlockSpec((1, Sk, D), lambda h: (h, 0, 0)),
        ],
        out_specs=pl.BlockSpec((1, Sq, Sk), lambda h: (h, 0, 0)),
        out_shape=jax.ShapeDtypeStruct((H, Sq, Sk), jnp.float32),
        scratch_shapes=[
            pltpu.VMEM((Sq, Sk), jnp.int32),
            pltpu.SemaphoreType.DMA,
        ],
        compiler_params=pltpu.CompilerParams(
            dimension_semantics=("arbitrary",),
        ),
    )(m2, a3, b3)
    return out.reshape(B, H, Sq, Sk)


# speedup vs baseline: 1.0369x; 1.0040x over previous
import functools

import jax
import jax.numpy as jnp
from jax.experimental import pallas as pl
from jax.experimental.pallas import tpu as pltpu


def _body(m_ref, a_ref, b_ref, o_ref):
    q = pl.program_id(1)
    TQ = a_ref.shape[1]
    av = a_ref[0]
    bv = b_ref[0]
    acc = jax.lax.dot_general(
        av, bv, (((1,), (1,)), ((), ())), preferred_element_type=jnp.float32
    )
    m = m_ref[pl.ds(q * TQ, TQ), :]
    o_ref[0] = jnp.where(m != 0, acc, jnp.float32(0.0))


@jax.jit
def kernel(a, b, mask):
    B, H, Sq, D = a.shape
    Sk = b.shape[2]
    TQ = 2048
    nq = Sq // TQ

    a3 = a.reshape(H, Sq, D)
    b3 = b.reshape(H, Sk, D)
    m2 = mask.reshape(Sq, Sk)

    out = pl.pallas_call(
        _body,
        grid=(H, nq),
        in_specs=[
            pl.BlockSpec((Sq, Sk), lambda h, q: (0, 0)),
            pl.BlockSpec((1, TQ, D), lambda h, q: (h, q, 0)),
            pl.BlockSpec((1, Sk, D), lambda h, q: (h, 0, 0)),
        ],
        out_specs=pl.BlockSpec((1, TQ, Sk), lambda h, q: (h, q, 0)),
        out_shape=jax.ShapeDtypeStruct((H, Sq, Sk), jnp.float32),
        compiler_params=pltpu.CompilerParams(
            dimension_semantics=("parallel", "parallel"),
        ),
    )(m2, a3, b3)
    return out.reshape(B, H, Sq, Sk)


# speedup vs baseline: 1.0373x; 1.0004x over previous
import jax
import jax.numpy as jnp
from jax.experimental import pallas as pl
from jax.experimental.pallas import tpu as pltpu


def _body(m_hbm, a_hbm, b_hbm, o_hbm,
          m_vmem, a_vmem, b_vmem, o_vmem,
          m_sem, a_sem, b_sem, o_sem):
    H, Sq, D = a_hbm.shape
    Sk = b_hbm.shape[1]
    HQ = Sq // 2
    NJ = 2 * H

    def mask_copy(half):
        return pltpu.make_async_copy(
            m_hbm.at[pl.ds(half * HQ, HQ), :],
            m_vmem.at[pl.ds(half * HQ, HQ), :],
            m_sem.at[half],
        )

    def a_copy(j):
        return pltpu.make_async_copy(
            a_hbm.at[j // 2, pl.ds((j % 2) * HQ, HQ), :],
            a_vmem.at[j % 2],
            a_sem.at[j % 2],
        )

    def b_copy(h):
        return pltpu.make_async_copy(b_hbm.at[h], b_vmem.at[h % 2], b_sem.at[h % 2])

    def out_copy(j, buf):
        return pltpu.make_async_copy(
            o_vmem.at[buf],
            o_hbm.at[j // 2, pl.ds((j % 2) * HQ, HQ), :],
            o_sem.at[buf],
        )

    a_copy(0).start()
    b_copy(0).start()
    mask_copy(0).start()
    mask_copy(1).start()
    a_copy(1).start()

    for h in range(H):
        b_copy(h).wait()
        if h + 1 < H:
            b_copy(h + 1).start()
        bv = b_vmem[h % 2]
        for half in range(2):
            j = 2 * h + half
            buf = j % 2
            a_copy(j).wait()
            if h == 0:
                mask_copy(half).wait()
            if j >= 2:
                out_copy(j - 2, buf).wait()
            av = a_vmem[j % 2]
            acc = jax.lax.dot_general(
                av, bv, (((1,), (1,)), ((), ())),
                preferred_element_type=jnp.float32,
            )
            if j + 2 < NJ:
                a_copy(j + 2).start()
            m = m_vmem[pl.ds(half * HQ, HQ), :]
            o_vmem[buf] = jnp.where(m != 0, acc, jnp.float32(0.0))
            out_copy(j, buf).start()

    out_copy(NJ - 2, 0).wait()
    out_copy(NJ - 1, 1).wait()


@jax.jit
def kernel(a, b, mask):
    B, H, Sq, D = a.shape
    Sk = b.shape[2]
    HQ = Sq // 2

    a3 = a.reshape(H, Sq, D)
    b3 = b.reshape(H, Sk, D)
    m2 = mask.reshape(Sq, Sk)

    out = pl.pallas_call(
        _body,
        in_specs=[
            pl.BlockSpec(memory_space=pl.ANY),
            pl.BlockSpec(memory_space=pl.ANY),
            pl.BlockSpec(memory_space=pl.ANY),
        ],
        out_specs=pl.BlockSpec(memory_space=pl.ANY),
        out_shape=jax.ShapeDtypeStruct((H, Sq, Sk), jnp.float32),
        scratch_shapes=[
            pltpu.VMEM((Sq, Sk), jnp.int32),
            pltpu.VMEM((2, HQ, D), jnp.float32),
            pltpu.VMEM((2, Sk, D), jnp.float32),
            pltpu.VMEM((2, HQ, Sk), jnp.float32),
            pltpu.SemaphoreType.DMA((2,)),
            pltpu.SemaphoreType.DMA((2,)),
            pltpu.SemaphoreType.DMA((2,)),
            pltpu.SemaphoreType.DMA((2,)),
        ],
    )(m2, a3, b3)
    return out.reshape(B, H, Sq, Sk)
